# Initial kernel scaffold; baseline (speedup 1.0000x reference)
import jax
import jax.numpy as jnp
from jax.experimental import pallas as pl


def kernel(ploc, plabel, dboxes):
    raise NotImplementedError("write your pallas kernel here")


# speedup vs baseline: 11.1486x; 11.1486x over previous
import jax
import jax.numpy as jnp
from jax import lax
from jax.experimental import pallas as pl
from jax.experimental.pallas import tpu as pltpu
from jax.experimental.pallas import tpu_sc as plsc

SCALE_XY = 0.1
SCALE_WH = 0.2
CRITERIA = 0.5
MAX_OUT = 200
SCORE_THRESH = 0.05
NCLS = 81
CM = NCLS - 1
NA = 15130
NW = 32
NA_PAD = 15360
K_MAX = 1024
PICK_PAD = 256
L = 16
NEG = -3.0e38
BIGF = 3.0e38


def _tc_dense(plabel_ref, ploc_ref, dbox_ref, probs_ref, geom_ref):
    z = plabel_ref[...]
    m = jnp.max(z, axis=0, keepdims=True)
    e = jnp.exp(z - m)
    denom = jnp.sum(e, axis=0, keepdims=True)
    probs_ref[...] = e[1:, :] / denom

    lx = ploc_ref[0:1, :] * SCALE_XY
    ly = ploc_ref[1:2, :] * SCALE_XY
    lw = ploc_ref[2:3, :] * SCALE_WH
    lh = ploc_ref[3:4, :] * SCALE_WH
    dx = dbox_ref[0:1, :]
    dy = dbox_ref[1:2, :]
    dw = dbox_ref[2:3, :]
    dh = dbox_ref[3:4, :]
    cx = lx * dw + dx
    cy = ly * dh + dy
    pw = jnp.exp(lw) * dw
    ph = jnp.exp(lh) * dh
    x1 = cx - 0.5 * pw
    y1 = cy - 0.5 * ph
    x2 = cx + 0.5 * pw
    y2 = cy + 0.5 * ph
    area = (x2 - x1) * (y2 - y1)
    zero = jnp.zeros_like(x1)
    geom_ref[...] = jnp.concatenate(
        [x1, y1, x2, y2, area, zero, zero, zero], axis=0)


def _perm(v, idx):
    return v.at[idx].get(mode="promise_in_bounds")


def _splat_max(v, ji):
    for d in (8, 4, 2, 1):
        v = jnp.maximum(v, _perm(v, ji ^ d))
    return v


def _splat_min(v, ji):
    for d in (8, 4, 2, 1):
        v = jnp.minimum(v, _perm(v, ji ^ d))
    return v


def _prefix_sum(v, ji):
    for d in (1, 2, 4, 8):
        sh = _perm(v, jnp.maximum(ji - d, 0))
        v = v + jnp.where(ji >= d, sh, 0.0)
    return v


def _sc_nms(probs_hbm, geom_hbm, psc_hbm, pan_hbm,
            probs_v, x1_v, y1_v, x2_v, y2_v, ar_v,
            csc, can, cx1, cy1, cx2, cy2, car, osc, oan):
    wid = lax.axis_index("s") * 2 + lax.axis_index("c")
    ji = lax.broadcasted_iota(jnp.int32, (L,), 0)
    jf = ji.astype(jnp.float32)
    lane0 = ji == 0
    l15 = jnp.broadcast_to(jnp.int32(15), (L,))

    pltpu.sync_copy(geom_hbm.at[0], x1_v)
    pltpu.sync_copy(geom_hbm.at[1], y1_v)
    pltpu.sync_copy(geom_hbm.at[2], x2_v)
    pltpu.sync_copy(geom_hbm.at[3], y2_v)
    pltpu.sync_copy(geom_hbm.at[4], ar_v)

    def process_class(c):
        pltpu.sync_copy(probs_hbm.at[c], probs_v)

        zf = jnp.zeros((L,), jnp.float32)
        zi = jnp.zeros((L,), jnp.int32)

        def zero_body(g, _):
            csc[pl.ds(g * L, L)] = zf
            can[pl.ds(g * L, L)] = zi
            return 0
        lax.fori_loop(0, (K_MAX + L) // L, zero_body, 0)

        def comp_body(g, off_v):
            v = probs_v[pl.ds(g * L, L)]
            m = v > SCORE_THRESH
            pfx = _prefix_sum(jnp.where(m, 1.0, 0.0), ji)
            idx = (off_v + pfx).astype(jnp.int32) - 1
            m2 = m & (idx < K_MAX)
            plsc.store_scatter(csc, [idx], v, mask=m2)
            plsc.store_scatter(can, [idx], g * L + ji, mask=m2)
            return off_v + _perm(pfx, l15)
        off_v = lax.fori_loop(0, NA_PAD // L, comp_body, zf)

        k_v = jnp.minimum(off_v.astype(jnp.int32), K_MAX)
        ng_v = (k_v + (L - 1)) >> 4
        ngroups = ng_v[0]

        def gather_body(g, carry):
            lanemax, lanegrp = carry
            idxv = can[pl.ds(g * L, L)]
            cx1[pl.ds(g * L, L)] = plsc.load_gather(x1_v, [idxv])
            cy1[pl.ds(g * L, L)] = plsc.load_gather(y1_v, [idxv])
            cx2[pl.ds(g * L, L)] = plsc.load_gather(x2_v, [idxv])
            cy2[pl.ds(g * L, L)] = plsc.load_gather(y2_v, [idxv])
            car[pl.ds(g * L, L)] = plsc.load_gather(ar_v, [idxv])
            v = csc[pl.ds(g * L, L)]
            upd = v > lanemax
            gf = jnp.broadcast_to(g, (L,)).astype(jnp.float32)
            lanemax = jnp.where(upd, v, lanemax)
            lanegrp = jnp.where(upd, gf, lanegrp)
            return lanemax, lanegrp
        lanemax, lanegrp = lax.fori_loop(0, ngroups, gather_body, (zf, zf))

        def pick_body(i, carry):
            lanemax, lanegrp = carry
            gmax = _splat_max(lanemax, ji)
            alive = gmax > 0.0
            candidx = jnp.where(lanemax == gmax, lanegrp * L + jf,
                                jnp.broadcast_to(jnp.float32(BIGF), (L,)))
            posf = _splat_min(candidx, ji)
            posi = jnp.where(alive, posf, 0.0).astype(jnp.int32)

            anchor_v = plsc.load_gather(can, [posi])
            px1 = plsc.load_gather(cx1, [posi])
            py1 = plsc.load_gather(cy1, [posi])
            px2 = plsc.load_gather(cx2, [posi])
            py2 = plsc.load_gather(cy2, [posi])
            par = plsc.load_gather(car, [posi])

            ivec = jnp.broadcast_to(i, (L,))
            plsc.store_scatter(osc, [ivec],
                               jnp.where(alive, gmax, 0.0), mask=lane0)
            plsc.store_scatter(oan, [ivec],
                               jnp.where(alive, anchor_v, 0), mask=lane0)

            def supp_body(g, carry2):
                lm, lg = carry2
                sl = pl.ds(g * L, L)
                v = csc[sl]
                ltx = jnp.maximum(px1, cx1[sl])
                lty = jnp.maximum(py1, cy1[sl])
                rbx = jnp.minimum(px2, cx2[sl])
                rby = jnp.minimum(py2, cy2[sl])
                w = jnp.maximum(rbx - ltx, 0.0)
                h = jnp.maximum(rby - lty, 0.0)
                inter = w * h
                iou = inter / (((par + car[sl]) - inter) + 1e-8)
                kill = (iou >= CRITERIA) & alive
                kill = kill | (g * L + ji == posi)
                v2 = jnp.where(kill, 0.0, v)
                csc[sl] = v2
                upd = v2 > lm
                gf = jnp.broadcast_to(g, (L,)).astype(jnp.float32)
                lm = jnp.where(upd, v2, lm)
                lg = jnp.where(upd, gf, lg)
                return lm, lg
            return lax.fori_loop(0, ngroups, supp_body, (zf, zf))
        lax.fori_loop(0, MAX_OUT, pick_body, (lanemax, lanegrp))

        pltpu.sync_copy(osc, psc_hbm.at[c])
        pltpu.sync_copy(oan, pan_hbm.at[c])

    for k in range(3):
        c = wid + k * NW

        @pl.when(c < CM)
        def _():
            process_class(c)


def _sc_merge(psc_hbm, pan_hbm, geom_hbm, obox_hbm, olab_hbm, osc_hbm,
              psc_v, pan_v, x1_v, y1_v, x2_v, y2_v,
              heads_v, ptr_v, obox_v, olab_v, osc_v):
    wid = lax.axis_index("s") * 2 + lax.axis_index("c")
    ji = lax.broadcasted_iota(jnp.int32, (L,), 0)
    jf = ji.astype(jnp.float32)
    lane0 = ji == 0
    NG = 6

    @pl.when(wid == 0)
    def _():
        pltpu.sync_copy(psc_hbm, psc_v)
        pltpu.sync_copy(pan_hbm, pan_v)
        pltpu.sync_copy(geom_hbm.at[0], x1_v)
        pltpu.sync_copy(geom_hbm.at[1], y1_v)
        pltpu.sync_copy(geom_hbm.at[2], x2_v)
        pltpu.sync_copy(geom_hbm.at[3], y2_v)

        zi = jnp.zeros((L,), jnp.int32)
        for g in range(NG):
            cls = g * L + ji
            valid = cls < CM
            clsc = jnp.minimum(cls, CM - 1)
            h = plsc.load_gather(psc_v, [clsc, zi])
            heads_v[pl.ds(g * L, L)] = jnp.where(
                valid, h, jnp.broadcast_to(jnp.float32(NEG), (L,)))
            ptr_v[pl.ds(g * L, L)] = zi

        def merge_body(i, _):
            lanemax = jnp.broadcast_to(jnp.float32(NEG), (L,))
            lanecls = jnp.zeros((L,), jnp.float32)
            for g in range(NG):
                v = heads_v[pl.ds(g * L, L)]
                upd = v > lanemax
                lanemax = jnp.where(upd, v, lanemax)
                lanecls = jnp.where(upd, g * L + jf, lanecls)
            gmax = _splat_max(lanemax, ji)
            candcls = jnp.where(lanemax == gmax, lanecls,
                                jnp.broadcast_to(jnp.float32(BIGF), (L,)))
            clsi = _splat_min(candcls, ji).astype(jnp.int32)

            p_v = plsc.load_gather(ptr_v, [clsi])
            anchor_v = plsc.load_gather(pan_v, [clsi, p_v])
            bx1 = plsc.load_gather(x1_v, [anchor_v])
            by1 = plsc.load_gather(y1_v, [anchor_v])
            bx2 = plsc.load_gather(x2_v, [anchor_v])
            by2 = plsc.load_gather(y2_v, [anchor_v])

            bval = jnp.where(ji == 0, bx1,
                             jnp.where(ji == 1, by1,
                                       jnp.where(ji == 2, bx2, by2)))
            plsc.store_scatter(obox_v, [4 * i + ji], bval, mask=ji < 4)
            ivec = jnp.broadcast_to(i, (L,))
            plsc.store_scatter(olab_v, [ivec], clsi + 1, mask=lane0)
            plsc.store_scatter(osc_v, [ivec], gmax, mask=lane0)

            p1 = p_v + 1
            plsc.store_scatter(ptr_v, [clsi], p1, mask=lane0)
            nh = plsc.load_gather(
                psc_v, [clsi, jnp.minimum(p1, MAX_OUT - 1)])
            nh = jnp.where(p1 < MAX_OUT, nh,
                           jnp.broadcast_to(jnp.float32(NEG), (L,)))
            plsc.store_scatter(heads_v, [clsi], nh, mask=lane0)
            return 0
        lax.fori_loop(0, MAX_OUT, merge_body, 0)

        pltpu.sync_copy(obox_v, obox_hbm)
        pltpu.sync_copy(olab_v, olab_hbm)
        pltpu.sync_copy(osc_v, osc_hbm)


def kernel(ploc, plabel, dboxes):
    assert ploc.shape == (1, 4, NA) and plabel.shape == (1, NCLS, NA)
    pad = NA_PAD - NA
    plabel_p = jnp.pad(plabel[0], ((0, 0), (0, pad)))
    ploc_p = jnp.pad(ploc[0], ((0, 0), (0, pad)))
    dbox_p = jnp.pad(dboxes.T, ((0, 0), (0, pad)))

    probs, geom = pl.pallas_call(
        _tc_dense,
        out_shape=(
            jax.ShapeDtypeStruct((CM, NA_PAD), jnp.float32),
            jax.ShapeDtypeStruct((8, NA_PAD), jnp.float32),
        ),
    )(plabel_p, ploc_p, dbox_p)

    mesh = plsc.VectorSubcoreMesh(core_axis_name="c", subcore_axis_name="s",
                                  num_cores=2, num_subcores=16)

    nms = pl.kernel(
        _sc_nms, mesh=mesh,
        compiler_params=pltpu.CompilerParams(needs_layout_passes=False),
        out_type=(
            jax.ShapeDtypeStruct((CM, PICK_PAD), jnp.float32),
            jax.ShapeDtypeStruct((CM, PICK_PAD), jnp.int32),
        ),
        scratch_types=[
            pltpu.VMEM((NA_PAD,), jnp.float32),
            pltpu.VMEM((NA_PAD,), jnp.float32),
            pltpu.VMEM((NA_PAD,), jnp.float32),
            pltpu.VMEM((NA_PAD,), jnp.float32),
            pltpu.VMEM((NA_PAD,), jnp.float32),
            pltpu.VMEM((NA_PAD,), jnp.float32),
            pltpu.VMEM((K_MAX + L,), jnp.float32),
            pltpu.VMEM((K_MAX + L,), jnp.int32),
            pltpu.VMEM((K_MAX + L,), jnp.float32),
            pltpu.VMEM((K_MAX + L,), jnp.float32),
            pltpu.VMEM((K_MAX + L,), jnp.float32),
            pltpu.VMEM((K_MAX + L,), jnp.float32),
            pltpu.VMEM((K_MAX + L,), jnp.float32),
            pltpu.VMEM((PICK_PAD,), jnp.float32),
            pltpu.VMEM((PICK_PAD,), jnp.int32),
        ],
    )
    pick_sc, pick_an = nms(probs, geom)

    merge = pl.kernel(
        _sc_merge, mesh=mesh,
        compiler_params=pltpu.CompilerParams(needs_layout_passes=False),
        out_type=(
            jax.ShapeDtypeStruct((4 * MAX_OUT,), jnp.float32),
            jax.ShapeDtypeStruct((MAX_OUT,), jnp.int32),
            jax.ShapeDtypeStruct((MAX_OUT,), jnp.float32),
        ),
        scratch_types=[
            pltpu.VMEM((CM, PICK_PAD), jnp.float32),
            pltpu.VMEM((CM, PICK_PAD), jnp.int32),
            pltpu.VMEM((NA_PAD,), jnp.float32),
            pltpu.VMEM((NA_PAD,), jnp.float32),
            pltpu.VMEM((NA_PAD,), jnp.float32),
            pltpu.VMEM((NA_PAD,), jnp.float32),
            pltpu.VMEM((96,), jnp.float32),
            pltpu.VMEM((96,), jnp.int32),
            pltpu.VMEM((4 * MAX_OUT,), jnp.float32),
            pltpu.VMEM((MAX_OUT,), jnp.int32),
            pltpu.VMEM((MAX_OUT,), jnp.float32),
        ],
    )
    obox, olab, osc = merge(pick_sc, pick_an, geom)

    return (obox.reshape(1, MAX_OUT, 4), olab.reshape(1, MAX_OUT),
            osc.reshape(1, MAX_OUT))


# speedup vs baseline: 11.6806x; 1.0477x over previous
import jax
import jax.numpy as jnp
from jax import lax
from jax.experimental import pallas as pl
from jax.experimental.pallas import tpu as pltpu
from jax.experimental.pallas import tpu_sc as plsc

SCALE_XY = 0.1
SCALE_WH = 0.2
CRITERIA = 0.5
MAX_OUT = 200
SCORE_THRESH = 0.05
NCLS = 81
CM = NCLS - 1
NA = 15130
NW = 32
NA_PAD = 15360
K_MAX = 1024
PICK_PAD = 256
L = 16
NEG = -3.0e38
BIGF = 3.0e38


def _tc_dense(plabel_ref, ploc_ref, dbox_ref, probs_ref, geom_ref):
    z = plabel_ref[...]
    m = jnp.max(z, axis=0, keepdims=True)
    e = jnp.exp(z - m)
    denom = jnp.sum(e, axis=0, keepdims=True)
    probs_ref[...] = e[1:, :] / denom

    lx = ploc_ref[0:1, :] * SCALE_XY
    ly = ploc_ref[1:2, :] * SCALE_XY
    lw = ploc_ref[2:3, :] * SCALE_WH
    lh = ploc_ref[3:4, :] * SCALE_WH
    dx = dbox_ref[0:1, :]
    dy = dbox_ref[1:2, :]
    dw = dbox_ref[2:3, :]
    dh = dbox_ref[3:4, :]
    cx = lx * dw + dx
    cy = ly * dh + dy
    pw = jnp.exp(lw) * dw
    ph = jnp.exp(lh) * dh
    x1 = cx - 0.5 * pw
    y1 = cy - 0.5 * ph
    x2 = cx + 0.5 * pw
    y2 = cy + 0.5 * ph
    area = (x2 - x1) * (y2 - y1)
    zero = jnp.zeros_like(x1)
    geom_ref[...] = jnp.concatenate(
        [x1, y1, x2, y2, area, zero, zero, zero], axis=0)


def _perm(v, idx):
    return v.at[idx].get(mode="promise_in_bounds")


def _splat_max(v, ji):
    for d in (8, 4, 2, 1):
        v = jnp.maximum(v, _perm(v, ji ^ d))
    return v


def _splat_min(v, ji):
    for d in (8, 4, 2, 1):
        v = jnp.minimum(v, _perm(v, ji ^ d))
    return v


def _prefix_sum(v, ji):
    for d in (1, 2, 4, 8):
        sh = _perm(v, jnp.maximum(ji - d, 0))
        v = v + jnp.where(ji >= d, sh, 0.0)
    return v


def _sc_nms(probs_hbm, geom_hbm, psc_hbm, pan_hbm,
            probs_v, x1_v, y1_v, x2_v, y2_v, ar_v,
            csc, can, cx1, cy1, cx2, cy2, car, osc, oan):
    wid = lax.axis_index("s") * 2 + lax.axis_index("c")
    ji = lax.broadcasted_iota(jnp.int32, (L,), 0)
    jf = ji.astype(jnp.float32)
    lane0 = ji == 0
    l15 = jnp.broadcast_to(jnp.int32(15), (L,))

    pltpu.sync_copy(geom_hbm.at[0], x1_v)
    pltpu.sync_copy(geom_hbm.at[1], y1_v)
    pltpu.sync_copy(geom_hbm.at[2], x2_v)
    pltpu.sync_copy(geom_hbm.at[3], y2_v)
    pltpu.sync_copy(geom_hbm.at[4], ar_v)

    def process_class(c):
        pltpu.sync_copy(probs_hbm.at[c], probs_v)

        zf = jnp.zeros((L,), jnp.float32)
        zi = jnp.zeros((L,), jnp.int32)

        def zero_body(g, _):
            csc[pl.ds(g * L, L)] = zf
            can[pl.ds(g * L, L)] = zi
            return 0
        lax.fori_loop(0, (K_MAX + 2 * L) // L, zero_body, 0)

        def comp_body(g, off_v):
            vA = probs_v[pl.ds((2 * g) * L, L)]
            vB = probs_v[pl.ds((2 * g + 1) * L, L)]
            mA = vA > SCORE_THRESH
            mB = vB > SCORE_THRESH
            pfxA = _prefix_sum(jnp.where(mA, 1.0, 0.0), ji)
            pfxB = _prefix_sum(jnp.where(mB, 1.0, 0.0), ji)
            totA = _perm(pfxA, l15)
            idxA = (off_v + pfxA).astype(jnp.int32) - 1
            idxB = (off_v + totA + pfxB).astype(jnp.int32) - 1
            m2A = mA & (idxA < K_MAX)
            m2B = mB & (idxB < K_MAX)
            plsc.store_scatter(csc, [idxA], vA, mask=m2A)
            plsc.store_scatter(can, [idxA], (2 * g) * L + ji, mask=m2A)
            plsc.store_scatter(csc, [idxB], vB, mask=m2B)
            plsc.store_scatter(can, [idxB], (2 * g + 1) * L + ji, mask=m2B)
            return off_v + totA + _perm(pfxB, l15)
        off_v = lax.fori_loop(0, NA_PAD // (2 * L), comp_body, zf)

        k_v = jnp.minimum(off_v.astype(jnp.int32), K_MAX)
        ng2_v = (k_v + (2 * L - 1)) >> 5
        ng2 = ng2_v[0]

        def gather_body(g, _):
            idxv = can[pl.ds(g * L, L)]
            cx1[pl.ds(g * L, L)] = plsc.load_gather(x1_v, [idxv])
            cy1[pl.ds(g * L, L)] = plsc.load_gather(y1_v, [idxv])
            cx2[pl.ds(g * L, L)] = plsc.load_gather(x2_v, [idxv])
            cy2[pl.ds(g * L, L)] = plsc.load_gather(y2_v, [idxv])
            car[pl.ds(g * L, L)] = plsc.load_gather(ar_v, [idxv])
            return 0
        lax.fori_loop(0, 2 * ng2, gather_body, 0)

        def init_body(g, carry):
            lmA, lgA, lmB, lgB = carry
            vA = csc[pl.ds((2 * g) * L, L)]
            vB = csc[pl.ds((2 * g + 1) * L, L)]
            gfA = jnp.broadcast_to(2 * g, (L,)).astype(jnp.float32)
            gfB = jnp.broadcast_to(2 * g + 1, (L,)).astype(jnp.float32)
            uA = vA > lmA
            uB = vB > lmB
            lmA = jnp.where(uA, vA, lmA)
            lgA = jnp.where(uA, gfA, lgA)
            lmB = jnp.where(uB, vB, lmB)
            lgB = jnp.where(uB, gfB, lgB)
            return lmA, lgA, lmB, lgB
        carry0 = lax.fori_loop(0, ng2, init_body, (zf, zf, zf, zf))

        def pick_body(i, carry):
            lmA, lgA, lmB, lgB = carry
            gmax = _splat_max(jnp.maximum(lmA, lmB), ji)
            alive = gmax > 0.0
            bigv = jnp.broadcast_to(jnp.float32(BIGF), (L,))
            cA = jnp.where(lmA == gmax, lgA * L + jf, bigv)
            cB = jnp.where(lmB == gmax, lgB * L + jf, bigv)
            posf = _splat_min(jnp.minimum(cA, cB), ji)
            posi = jnp.where(alive, posf, 0.0).astype(jnp.int32)

            anchor_v = plsc.load_gather(can, [posi])
            px1 = plsc.load_gather(cx1, [posi])
            py1 = plsc.load_gather(cy1, [posi])
            px2 = plsc.load_gather(cx2, [posi])
            py2 = plsc.load_gather(cy2, [posi])
            par = plsc.load_gather(car, [posi])

            ivec = jnp.broadcast_to(i, (L,))
            plsc.store_scatter(osc, [ivec],
                               jnp.where(alive, gmax, 0.0), mask=lane0)
            plsc.store_scatter(oan, [ivec],
                               jnp.where(alive, anchor_v, 0), mask=lane0)

            def one_group(g, lm, lg):
                sl = pl.ds(g * L, L)
                v = csc[sl]
                ltx = jnp.maximum(px1, cx1[sl])
                lty = jnp.maximum(py1, cy1[sl])
                rbx = jnp.minimum(px2, cx2[sl])
                rby = jnp.minimum(py2, cy2[sl])
                w = jnp.maximum(rbx - ltx, 0.0)
                h = jnp.maximum(rby - lty, 0.0)
                inter = w * h
                iou = inter / (((par + car[sl]) - inter) + 1e-8)
                kill = (iou >= CRITERIA) & alive
                kill = kill | (g * L + ji == posi)
                v2 = jnp.where(kill, 0.0, v)
                csc[sl] = v2
                upd = v2 > lm
                gf = jnp.broadcast_to(g, (L,)).astype(jnp.float32)
                lm = jnp.where(upd, v2, lm)
                lg = jnp.where(upd, gf, lg)
                return lm, lg

            def supp_body(g, carry2):
                lmA, lgA, lmB, lgB = carry2
                lmA, lgA = one_group(2 * g, lmA, lgA)
                lmB, lgB = one_group(2 * g + 1, lmB, lgB)
                return lmA, lgA, lmB, lgB
            return lax.fori_loop(0, ng2, supp_body,
                                 (zf, zf, zf, zf))
        lax.fori_loop(0, MAX_OUT, pick_body, carry0)

        pltpu.sync_copy(osc, psc_hbm.at[c])
        pltpu.sync_copy(oan, pan_hbm.at[c])

    for k in range(3):
        c = wid + k * NW

        @pl.when(c < CM)
        def _():
            process_class(c)


def _sc_merge(psc_hbm, pan_hbm, geom_hbm, obox_hbm, olab_hbm, osc_hbm,
              psc_v, pan_v, x1_v, y1_v, x2_v, y2_v,
              heads_v, ptr_v, obox_v, olab_v, osc_v):
    wid = lax.axis_index("s") * 2 + lax.axis_index("c")
    ji = lax.broadcasted_iota(jnp.int32, (L,), 0)
    jf = ji.astype(jnp.float32)
    lane0 = ji == 0
    NG = 6

    @pl.when(wid == 0)
    def _():
        pltpu.sync_copy(psc_hbm, psc_v)
        pltpu.sync_copy(pan_hbm, pan_v)
        pltpu.sync_copy(geom_hbm.at[0], x1_v)
        pltpu.sync_copy(geom_hbm.at[1], y1_v)
        pltpu.sync_copy(geom_hbm.at[2], x2_v)
        pltpu.sync_copy(geom_hbm.at[3], y2_v)

        zi = jnp.zeros((L,), jnp.int32)
        for g in range(NG):
            cls = g * L + ji
            valid = cls < CM
            clsc = jnp.minimum(cls, CM - 1)
            h = plsc.load_gather(psc_v, [clsc, zi])
            heads_v[pl.ds(g * L, L)] = jnp.where(
                valid, h, jnp.broadcast_to(jnp.float32(NEG), (L,)))
            ptr_v[pl.ds(g * L, L)] = zi

        def merge_body(i, _):
            lanemax = jnp.broadcast_to(jnp.float32(NEG), (L,))
            lanecls = jnp.zeros((L,), jnp.float32)
            for g in range(NG):
                v = heads_v[pl.ds(g * L, L)]
                upd = v > lanemax
                lanemax = jnp.where(upd, v, lanemax)
                lanecls = jnp.where(upd, g * L + jf, lanecls)
            gmax = _splat_max(lanemax, ji)
            candcls = jnp.where(lanemax == gmax, lanecls,
                                jnp.broadcast_to(jnp.float32(BIGF), (L,)))
            clsi = _splat_min(candcls, ji).astype(jnp.int32)

            p_v = plsc.load_gather(ptr_v, [clsi])
            anchor_v = plsc.load_gather(pan_v, [clsi, p_v])
            bx1 = plsc.load_gather(x1_v, [anchor_v])
            by1 = plsc.load_gather(y1_v, [anchor_v])
            bx2 = plsc.load_gather(x2_v, [anchor_v])
            by2 = plsc.load_gather(y2_v, [anchor_v])

            bval = jnp.where(ji == 0, bx1,
                             jnp.where(ji == 1, by1,
                                       jnp.where(ji == 2, bx2, by2)))
            plsc.store_scatter(obox_v, [4 * i + ji], bval, mask=ji < 4)
            ivec = jnp.broadcast_to(i, (L,))
            plsc.store_scatter(olab_v, [ivec], clsi + 1, mask=lane0)
            plsc.store_scatter(osc_v, [ivec], gmax, mask=lane0)

            p1 = p_v + 1
            plsc.store_scatter(ptr_v, [clsi], p1, mask=lane0)
            nh = plsc.load_gather(
                psc_v, [clsi, jnp.minimum(p1, MAX_OUT - 1)])
            nh = jnp.where(p1 < MAX_OUT, nh,
                           jnp.broadcast_to(jnp.float32(NEG), (L,)))
            plsc.store_scatter(heads_v, [clsi], nh, mask=lane0)
            return 0
        lax.fori_loop(0, MAX_OUT, merge_body, 0)

        pltpu.sync_copy(obox_v, obox_hbm)
        pltpu.sync_copy(olab_v, olab_hbm)
        pltpu.sync_copy(osc_v, osc_hbm)


def kernel(ploc, plabel, dboxes):
    assert ploc.shape == (1, 4, NA) and plabel.shape == (1, NCLS, NA)
    pad = NA_PAD - NA
    plabel_p = jnp.pad(plabel[0], ((0, 0), (0, pad)))
    ploc_p = jnp.pad(ploc[0], ((0, 0), (0, pad)))
    dbox_p = jnp.pad(dboxes.T, ((0, 0), (0, pad)))

    probs, geom = pl.pallas_call(
        _tc_dense,
        out_shape=(
            jax.ShapeDtypeStruct((CM, NA_PAD), jnp.float32),
            jax.ShapeDtypeStruct((8, NA_PAD), jnp.float32),
        ),
    )(plabel_p, ploc_p, dbox_p)

    mesh = plsc.VectorSubcoreMesh(core_axis_name="c", subcore_axis_name="s",
                                  num_cores=2, num_subcores=16)

    nms = pl.kernel(
        _sc_nms, mesh=mesh,
        compiler_params=pltpu.CompilerParams(needs_layout_passes=False),
        out_type=(
            jax.ShapeDtypeStruct((CM, PICK_PAD), jnp.float32),
            jax.ShapeDtypeStruct((CM, PICK_PAD), jnp.int32),
        ),
        scratch_types=[
            pltpu.VMEM((NA_PAD,), jnp.float32),
            pltpu.VMEM((NA_PAD,), jnp.float32),
            pltpu.VMEM((NA_PAD,), jnp.float32),
            pltpu.VMEM((NA_PAD,), jnp.float32),
            pltpu.VMEM((NA_PAD,), jnp.float32),
            pltpu.VMEM((NA_PAD,), jnp.float32),
            pltpu.VMEM((K_MAX + 2 * L,), jnp.float32),
            pltpu.VMEM((K_MAX + 2 * L,), jnp.int32),
            pltpu.VMEM((K_MAX + 2 * L,), jnp.float32),
            pltpu.VMEM((K_MAX + 2 * L,), jnp.float32),
            pltpu.VMEM((K_MAX + 2 * L,), jnp.float32),
            pltpu.VMEM((K_MAX + 2 * L,), jnp.float32),
            pltpu.VMEM((K_MAX + 2 * L,), jnp.float32),
            pltpu.VMEM((PICK_PAD,), jnp.float32),
            pltpu.VMEM((PICK_PAD,), jnp.int32),
        ],
    )
    pick_sc, pick_an = nms(probs, geom)

    merge = pl.kernel(
        _sc_merge, mesh=mesh,
        compiler_params=pltpu.CompilerParams(needs_layout_passes=False),
        out_type=(
            jax.ShapeDtypeStruct((4 * MAX_OUT,), jnp.float32),
            jax.ShapeDtypeStruct((MAX_OUT,), jnp.int32),
            jax.ShapeDtypeStruct((MAX_OUT,), jnp.float32),
        ),
        scratch_types=[
            pltpu.VMEM((CM, PICK_PAD), jnp.float32),
            pltpu.VMEM((CM, PICK_PAD), jnp.int32),
            pltpu.VMEM((NA_PAD,), jnp.float32),
            pltpu.VMEM((NA_PAD,), jnp.float32),
            pltpu.VMEM((NA_PAD,), jnp.float32),
            pltpu.VMEM((NA_PAD,), jnp.float32),
            pltpu.VMEM((96,), jnp.float32),
            pltpu.VMEM((96,), jnp.int32),
            pltpu.VMEM((4 * MAX_OUT,), jnp.float32),
            pltpu.VMEM((MAX_OUT,), jnp.int32),
            pltpu.VMEM((MAX_OUT,), jnp.float32),
        ],
    )
    obox, olab, osc = merge(pick_sc, pick_an, geom)

    return (obox.reshape(1, MAX_OUT, 4), olab.reshape(1, MAX_OUT),
            osc.reshape(1, MAX_OUT))


# speedup vs baseline: 17.2133x; 1.4737x over previous
import jax
import jax.numpy as jnp
from jax import lax
from jax.experimental import pallas as pl
from jax.experimental.pallas import tpu as pltpu
from jax.experimental.pallas import tpu_sc as plsc

SCALE_XY = 0.1
SCALE_WH = 0.2
CRITERIA = 0.5
MAX_OUT = 200
SCORE_THRESH = 0.05
NCLS = 81
CM = NCLS - 1
NA = 15130
NW = 32
NA_PAD = 15360
K_MAX = 1024
PICK_PAD = 256
L = 16
NEG = -3.0e38
BIGF = 3.0e38


def _tc_dense(plabel_ref, ploc_ref, dbox_ref, probs_ref, geom_ref):
    z = plabel_ref[...]
    m = jnp.max(z, axis=0, keepdims=True)
    e = jnp.exp(z - m)
    denom = jnp.sum(e, axis=0, keepdims=True)
    probs_ref[...] = e[1:, :] / denom

    lx = ploc_ref[0:1, :] * SCALE_XY
    ly = ploc_ref[1:2, :] * SCALE_XY
    lw = ploc_ref[2:3, :] * SCALE_WH
    lh = ploc_ref[3:4, :] * SCALE_WH
    dx = dbox_ref[0:1, :]
    dy = dbox_ref[1:2, :]
    dw = dbox_ref[2:3, :]
    dh = dbox_ref[3:4, :]
    cx = lx * dw + dx
    cy = ly * dh + dy
    pw = jnp.exp(lw) * dw
    ph = jnp.exp(lh) * dh
    x1 = cx - 0.5 * pw
    y1 = cy - 0.5 * ph
    x2 = cx + 0.5 * pw
    y2 = cy + 0.5 * ph
    area = (x2 - x1) * (y2 - y1)
    zero = jnp.zeros_like(x1)
    geom_ref[...] = jnp.concatenate(
        [x1, y1, x2, y2, area, zero, zero, zero], axis=0)


def _perm(v, idx):
    return v.at[idx].get(mode="promise_in_bounds")


def _splat_max(v, ji):
    for d in (8, 4, 2, 1):
        v = jnp.maximum(v, _perm(v, ji ^ d))
    return v


def _splat_min(v, ji):
    for d in (8, 4, 2, 1):
        v = jnp.minimum(v, _perm(v, ji ^ d))
    return v


def _prefix_sum(v, ji):
    for d in (1, 2, 4, 8):
        sh = _perm(v, jnp.maximum(ji - d, 0))
        v = v + jnp.where(ji >= d, sh, 0.0)
    return v


def _sc_nms(probs_hbm, geom_hbm, psc_hbm, pan_hbm,
            probs_v, x1_v, y1_v, x2_v, y2_v, ar_v,
            csc, can, cx1, cy1, cx2, cy2, car, osc, oan):
    wid = lax.axis_index("s") * 2 + lax.axis_index("c")
    ji = lax.broadcasted_iota(jnp.int32, (L,), 0)
    jf = ji.astype(jnp.float32)
    lane0 = ji == 0
    l15 = jnp.broadcast_to(jnp.int32(15), (L,))

    pltpu.sync_copy(geom_hbm.at[0], x1_v)
    pltpu.sync_copy(geom_hbm.at[1], y1_v)
    pltpu.sync_copy(geom_hbm.at[2], x2_v)
    pltpu.sync_copy(geom_hbm.at[3], y2_v)
    pltpu.sync_copy(geom_hbm.at[4], ar_v)

    def process_class(c):
        pltpu.sync_copy(probs_hbm.at[c], probs_v)

        zf = jnp.zeros((L,), jnp.float32)
        zi = jnp.zeros((L,), jnp.int32)

        def zero_body(g, _):
            csc[pl.ds(g * L, L)] = zf
            can[pl.ds(g * L, L)] = zi
            return 0
        lax.fori_loop(0, (K_MAX + 2 * L) // L, zero_body, 0)

        def comp_body(g, off_v):
            vA = probs_v[pl.ds((2 * g) * L, L)]
            vB = probs_v[pl.ds((2 * g + 1) * L, L)]
            mA = vA > SCORE_THRESH
            mB = vB > SCORE_THRESH
            pfxA = _prefix_sum(jnp.where(mA, 1.0, 0.0), ji)
            pfxB = _prefix_sum(jnp.where(mB, 1.0, 0.0), ji)
            totA = _perm(pfxA, l15)
            idxA = (off_v + pfxA).astype(jnp.int32) - 1
            idxB = (off_v + totA + pfxB).astype(jnp.int32) - 1
            m2A = mA & (idxA < K_MAX)
            m2B = mB & (idxB < K_MAX)
            plsc.store_scatter(csc, [idxA], vA, mask=m2A)
            plsc.store_scatter(can, [idxA], (2 * g) * L + ji, mask=m2A)
            plsc.store_scatter(csc, [idxB], vB, mask=m2B)
            plsc.store_scatter(can, [idxB], (2 * g + 1) * L + ji, mask=m2B)
            return off_v + totA + _perm(pfxB, l15)
        off_v = lax.fori_loop(0, NA_PAD // (2 * L), comp_body, zf)

        k_v = jnp.minimum(off_v.astype(jnp.int32), K_MAX)
        ng2_v = (k_v + (2 * L - 1)) >> 5
        ng2 = ng2_v[0]

        def gather_body(g, _):
            idxv = can[pl.ds(g * L, L)]
            cx1[pl.ds(g * L, L)] = plsc.load_gather(x1_v, [idxv])
            cy1[pl.ds(g * L, L)] = plsc.load_gather(y1_v, [idxv])
            cx2[pl.ds(g * L, L)] = plsc.load_gather(x2_v, [idxv])
            cy2[pl.ds(g * L, L)] = plsc.load_gather(y2_v, [idxv])
            car[pl.ds(g * L, L)] = plsc.load_gather(ar_v, [idxv])
            return 0
        lax.fori_loop(0, 2 * ng2, gather_body, 0)

        def init_body(g, carry):
            lmA, lgA, lmB, lgB = carry
            vA = csc[pl.ds((2 * g) * L, L)]
            vB = csc[pl.ds((2 * g + 1) * L, L)]
            gfA = jnp.broadcast_to(2 * g, (L,)).astype(jnp.float32)
            gfB = jnp.broadcast_to(2 * g + 1, (L,)).astype(jnp.float32)
            uA = vA > lmA
            uB = vB > lmB
            lmA = jnp.where(uA, vA, lmA)
            lgA = jnp.where(uA, gfA, lgA)
            lmB = jnp.where(uB, vB, lmB)
            lgB = jnp.where(uB, gfB, lgB)
            return lmA, lgA, lmB, lgB
        carry0 = lax.fori_loop(0, ng2, init_body, (zf, zf, zf, zf))

        def pick_body(i, carry):
            lmA, lgA, lmB, lgB = carry
            gmax = _splat_max(jnp.maximum(lmA, lmB), ji)
            alive = gmax > 0.0
            bigv = jnp.broadcast_to(jnp.float32(BIGF), (L,))
            cA = jnp.where(lmA == gmax, lgA * L + jf, bigv)
            cB = jnp.where(lmB == gmax, lgB * L + jf, bigv)
            posf = _splat_min(jnp.minimum(cA, cB), ji)
            posi = jnp.where(alive, posf, 0.0).astype(jnp.int32)

            anchor_v = plsc.load_gather(can, [posi])
            px1 = plsc.load_gather(cx1, [posi])
            py1 = plsc.load_gather(cy1, [posi])
            px2 = plsc.load_gather(cx2, [posi])
            py2 = plsc.load_gather(cy2, [posi])
            par = plsc.load_gather(car, [posi])

            ivec = jnp.broadcast_to(i, (L,))
            plsc.store_scatter(osc, [ivec],
                               jnp.where(alive, gmax, 0.0), mask=lane0)
            plsc.store_scatter(oan, [ivec],
                               jnp.where(alive, anchor_v, 0), mask=lane0)

            def supp_body(g, carry2):
                lmA, lgA, lmB, lgB = carry2
                slA = pl.ds((2 * g) * L, L)
                slB = pl.ds((2 * g + 1) * L, L)
                vA = csc[slA]
                vB = csc[slB]
                ax1 = cx1[slA]
                bx1 = cx1[slB]
                ay1 = cy1[slA]
                by1 = cy1[slB]
                ax2 = cx2[slA]
                bx2 = cx2[slB]
                ay2 = cy2[slA]
                by2 = cy2[slB]
                aar = car[slA]
                bar = car[slB]
                wA = jnp.maximum(jnp.minimum(px2, ax2) - jnp.maximum(px1, ax1), 0.0)
                wB = jnp.maximum(jnp.minimum(px2, bx2) - jnp.maximum(px1, bx1), 0.0)
                hA = jnp.maximum(jnp.minimum(py2, ay2) - jnp.maximum(py1, ay1), 0.0)
                hB = jnp.maximum(jnp.minimum(py2, by2) - jnp.maximum(py1, by1), 0.0)
                inA = wA * hA
                inB = wB * hB
                iouA = inA / (((par + aar) - inA) + 1e-8)
                iouB = inB / (((par + bar) - inB) + 1e-8)
                kA = ((iouA >= CRITERIA) & alive) | ((2 * g) * L + ji == posi)
                kB = ((iouB >= CRITERIA) & alive) | ((2 * g + 1) * L + ji == posi)
                v2A = jnp.where(kA, 0.0, vA)
                v2B = jnp.where(kB, 0.0, vB)
                csc[slA] = v2A
                csc[slB] = v2B
                uA = v2A > lmA
                uB = v2B > lmB
                gfA = jnp.broadcast_to(2 * g, (L,)).astype(jnp.float32)
                gfB = jnp.broadcast_to(2 * g + 1, (L,)).astype(jnp.float32)
                lmA = jnp.where(uA, v2A, lmA)
                lgA = jnp.where(uA, gfA, lgA)
                lmB = jnp.where(uB, v2B, lmB)
                lgB = jnp.where(uB, gfB, lgB)
                return lmA, lgA, lmB, lgB
            return lax.fori_loop(0, ng2, supp_body,
                                 (zf, zf, zf, zf))
        lax.fori_loop(0, MAX_OUT, pick_body, carry0)

        pltpu.sync_copy(osc, psc_hbm.at[c])
        pltpu.sync_copy(oan, pan_hbm.at[c])

    for k in range(3):
        c = wid + k * NW

        @pl.when(c < CM)
        def _():
            process_class(c)


def _sc_merge(psc_hbm, pan_hbm, geom_hbm, obox_hbm, olab_hbm, osc_hbm,
              psc_v, pan_v, x1_v, y1_v, x2_v, y2_v,
              heads_v, ptr_v, obox_v, olab_v, osc_v):
    wid = lax.axis_index("s") * 2 + lax.axis_index("c")
    ji = lax.broadcasted_iota(jnp.int32, (L,), 0)
    jf = ji.astype(jnp.float32)
    lane0 = ji == 0
    NG = 6

    @pl.when(wid == 0)
    def _():
        pltpu.sync_copy(psc_hbm, psc_v)
        pltpu.sync_copy(pan_hbm, pan_v)
        pltpu.sync_copy(geom_hbm.at[0], x1_v)
        pltpu.sync_copy(geom_hbm.at[1], y1_v)
        pltpu.sync_copy(geom_hbm.at[2], x2_v)
        pltpu.sync_copy(geom_hbm.at[3], y2_v)

        zi = jnp.zeros((L,), jnp.int32)
        for g in range(NG):
            cls = g * L + ji
            valid = cls < CM
            clsc = jnp.minimum(cls, CM - 1)
            h = plsc.load_gather(psc_v, [clsc, zi])
            heads_v[pl.ds(g * L, L)] = jnp.where(
                valid, h, jnp.broadcast_to(jnp.float32(NEG), (L,)))
            ptr_v[pl.ds(g * L, L)] = zi

        def merge_body(i, _):
            lanemax = jnp.broadcast_to(jnp.float32(NEG), (L,))
            lanecls = jnp.zeros((L,), jnp.float32)
            for g in range(NG):
                v = heads_v[pl.ds(g * L, L)]
                upd = v > lanemax
                lanemax = jnp.where(upd, v, lanemax)
                lanecls = jnp.where(upd, g * L + jf, lanecls)
            gmax = _splat_max(lanemax, ji)
            candcls = jnp.where(lanemax == gmax, lanecls,
                                jnp.broadcast_to(jnp.float32(BIGF), (L,)))
            clsi = _splat_min(candcls, ji).astype(jnp.int32)

            p_v = plsc.load_gather(ptr_v, [clsi])
            anchor_v = plsc.load_gather(pan_v, [clsi, p_v])
            bx1 = plsc.load_gather(x1_v, [anchor_v])
            by1 = plsc.load_gather(y1_v, [anchor_v])
            bx2 = plsc.load_gather(x2_v, [anchor_v])
            by2 = plsc.load_gather(y2_v, [anchor_v])

            bval = jnp.where(ji == 0, bx1,
                             jnp.where(ji == 1, by1,
                                       jnp.where(ji == 2, bx2, by2)))
            plsc.store_scatter(obox_v, [4 * i + ji], bval, mask=ji < 4)
            ivec = jnp.broadcast_to(i, (L,))
            plsc.store_scatter(olab_v, [ivec], clsi + 1, mask=lane0)
            plsc.store_scatter(osc_v, [ivec], gmax, mask=lane0)

            p1 = p_v + 1
            plsc.store_scatter(ptr_v, [clsi], p1, mask=lane0)
            nh = plsc.load_gather(
                psc_v, [clsi, jnp.minimum(p1, MAX_OUT - 1)])
            nh = jnp.where(p1 < MAX_OUT, nh,
                           jnp.broadcast_to(jnp.float32(NEG), (L,)))
            plsc.store_scatter(heads_v, [clsi], nh, mask=lane0)
            return 0
        lax.fori_loop(0, MAX_OUT, merge_body, 0)

        pltpu.sync_copy(obox_v, obox_hbm)
        pltpu.sync_copy(olab_v, olab_hbm)
        pltpu.sync_copy(osc_v, osc_hbm)


def kernel(ploc, plabel, dboxes):
    assert ploc.shape == (1, 4, NA) and plabel.shape == (1, NCLS, NA)
    pad = NA_PAD - NA
    plabel_p = jnp.pad(plabel[0], ((0, 0), (0, pad)))
    ploc_p = jnp.pad(ploc[0], ((0, 0), (0, pad)))
    dbox_p = jnp.pad(dboxes.T, ((0, 0), (0, pad)))

    probs, geom = pl.pallas_call(
        _tc_dense,
        out_shape=(
            jax.ShapeDtypeStruct((CM, NA_PAD), jnp.float32),
            jax.ShapeDtypeStruct((8, NA_PAD), jnp.float32),
        ),
    )(plabel_p, ploc_p, dbox_p)

    mesh = plsc.VectorSubcoreMesh(core_axis_name="c", subcore_axis_name="s",
                                  num_cores=2, num_subcores=16)

    nms = pl.kernel(
        _sc_nms, mesh=mesh,
        compiler_params=pltpu.CompilerParams(needs_layout_passes=False),
        out_type=(
            jax.ShapeDtypeStruct((CM, PICK_PAD), jnp.float32),
            jax.ShapeDtypeStruct((CM, PICK_PAD), jnp.int32),
        ),
        scratch_types=[
            pltpu.VMEM((NA_PAD,), jnp.float32),
            pltpu.VMEM((NA_PAD,), jnp.float32),
            pltpu.VMEM((NA_PAD,), jnp.float32),
            pltpu.VMEM((NA_PAD,), jnp.float32),
            pltpu.VMEM((NA_PAD,), jnp.float32),
            pltpu.VMEM((NA_PAD,), jnp.float32),
            pltpu.VMEM((K_MAX + 2 * L,), jnp.float32),
            pltpu.VMEM((K_MAX + 2 * L,), jnp.int32),
            pltpu.VMEM((K_MAX + 2 * L,), jnp.float32),
            pltpu.VMEM((K_MAX + 2 * L,), jnp.float32),
            pltpu.VMEM((K_MAX + 2 * L,), jnp.float32),
            pltpu.VMEM((K_MAX + 2 * L,), jnp.float32),
            pltpu.VMEM((K_MAX + 2 * L,), jnp.float32),
            pltpu.VMEM((PICK_PAD,), jnp.float32),
            pltpu.VMEM((PICK_PAD,), jnp.int32),
        ],
    )
    pick_sc, pick_an = nms(probs, geom)

    merge = pl.kernel(
        _sc_merge, mesh=mesh,
        compiler_params=pltpu.CompilerParams(needs_layout_passes=False),
        out_type=(
            jax.ShapeDtypeStruct((4 * MAX_OUT,), jnp.float32),
            jax.ShapeDtypeStruct((MAX_OUT,), jnp.int32),
            jax.ShapeDtypeStruct((MAX_OUT,), jnp.float32),
        ),
        scratch_types=[
            pltpu.VMEM((CM, PICK_PAD), jnp.float32),
            pltpu.VMEM((CM, PICK_PAD), jnp.int32),
            pltpu.VMEM((NA_PAD,), jnp.float32),
            pltpu.VMEM((NA_PAD,), jnp.float32),
            pltpu.VMEM((NA_PAD,), jnp.float32),
            pltpu.VMEM((NA_PAD,), jnp.float32),
            pltpu.VMEM((96,), jnp.float32),
            pltpu.VMEM((96,), jnp.int32),
            pltpu.VMEM((4 * MAX_OUT,), jnp.float32),
            pltpu.VMEM((MAX_OUT,), jnp.int32),
            pltpu.VMEM((MAX_OUT,), jnp.float32),
        ],
    )
    obox, olab, osc = merge(pick_sc, pick_an, geom)

    return (obox.reshape(1, MAX_OUT, 4), olab.reshape(1, MAX_OUT),
            osc.reshape(1, MAX_OUT))


# speedup vs baseline: 24.0458x; 1.3969x over previous
import jax
import jax.numpy as jnp
from jax import lax
from jax.experimental import pallas as pl
from jax.experimental.pallas import tpu as pltpu
from jax.experimental.pallas import tpu_sc as plsc

SCALE_XY = 0.1
SCALE_WH = 0.2
CRITERIA = 0.5
MAX_OUT = 200
SCORE_THRESH = 0.05
NCLS = 81
CM = NCLS - 1
NA = 15130
NW = 32
NA_PAD = 15360
K_MAX = 1024
PICK_PAD = 256
L = 16
NEG = -3.0e38
BIGF = 3.0e38


def _tc_dense(plabel_ref, ploc_ref, dbox_ref, probs_ref, geom_ref):
    z = plabel_ref[...]
    m = jnp.max(z, axis=0, keepdims=True)
    e = jnp.exp(z - m)
    denom = jnp.sum(e, axis=0, keepdims=True)
    probs_ref[...] = e[1:, :] / denom

    lx = ploc_ref[0:1, :] * SCALE_XY
    ly = ploc_ref[1:2, :] * SCALE_XY
    lw = ploc_ref[2:3, :] * SCALE_WH
    lh = ploc_ref[3:4, :] * SCALE_WH
    dx = dbox_ref[0:1, :]
    dy = dbox_ref[1:2, :]
    dw = dbox_ref[2:3, :]
    dh = dbox_ref[3:4, :]
    cx = lx * dw + dx
    cy = ly * dh + dy
    pw = jnp.exp(lw) * dw
    ph = jnp.exp(lh) * dh
    x1 = cx - 0.5 * pw
    y1 = cy - 0.5 * ph
    x2 = cx + 0.5 * pw
    y2 = cy + 0.5 * ph
    area = (x2 - x1) * (y2 - y1)
    zero = jnp.zeros_like(x1)
    geom_ref[...] = jnp.concatenate(
        [x1, y1, x2, y2, area, zero, zero, zero], axis=0)


def _perm(v, idx):
    return v.at[idx].get(mode="promise_in_bounds")


def _splat_max(v, ji):
    for d in (8, 4, 2, 1):
        v = jnp.maximum(v, _perm(v, ji ^ d))
    return v


def _splat_min(v, ji):
    for d in (8, 4, 2, 1):
        v = jnp.minimum(v, _perm(v, ji ^ d))
    return v


def _prefix_sum(v, ji):
    for d in (1, 2, 4, 8):
        sh = _perm(v, jnp.maximum(ji - d, 0))
        v = v + jnp.where(ji >= d, sh, 0.0)
    return v


def _sc_nms(probs_hbm, geom_hbm, psc_hbm, pan_hbm,
            probs_v, x1_v, y1_v, x2_v, y2_v, ar_v,
            csc, can, cx1, cy1, cx2, cy2, car, osc, oan):
    wid = lax.axis_index("s") * 2 + lax.axis_index("c")
    ji = lax.broadcasted_iota(jnp.int32, (L,), 0)
    jf = ji.astype(jnp.float32)
    lane0 = ji == 0
    l15 = jnp.broadcast_to(jnp.int32(15), (L,))

    pltpu.sync_copy(geom_hbm.at[0], x1_v)
    pltpu.sync_copy(geom_hbm.at[1], y1_v)
    pltpu.sync_copy(geom_hbm.at[2], x2_v)
    pltpu.sync_copy(geom_hbm.at[3], y2_v)
    pltpu.sync_copy(geom_hbm.at[4], ar_v)

    def process_class(c):
        pltpu.sync_copy(probs_hbm.at[c], probs_v)

        zf = jnp.zeros((L,), jnp.float32)
        zi = jnp.zeros((L,), jnp.int32)
        U = 4

        def zero_body(g, _):
            csc[pl.ds(g * L, L)] = zf
            can[pl.ds(g * L, L)] = zi
            return 0
        lax.fori_loop(0, (K_MAX + U * L) // L, zero_body, 0)

        def comp_body(g, off_v):
            vA = probs_v[pl.ds((2 * g) * L, L)]
            vB = probs_v[pl.ds((2 * g + 1) * L, L)]
            mA = vA > SCORE_THRESH
            mB = vB > SCORE_THRESH
            pfxA = _prefix_sum(jnp.where(mA, 1.0, 0.0), ji)
            pfxB = _prefix_sum(jnp.where(mB, 1.0, 0.0), ji)
            totA = _perm(pfxA, l15)
            idxA = (off_v + pfxA).astype(jnp.int32) - 1
            idxB = (off_v + totA + pfxB).astype(jnp.int32) - 1
            m2A = mA & (idxA < K_MAX)
            m2B = mB & (idxB < K_MAX)
            plsc.store_scatter(csc, [idxA], vA, mask=m2A)
            plsc.store_scatter(can, [idxA], (2 * g) * L + ji, mask=m2A)
            plsc.store_scatter(csc, [idxB], vB, mask=m2B)
            plsc.store_scatter(can, [idxB], (2 * g + 1) * L + ji, mask=m2B)
            return off_v + totA + _perm(pfxB, l15)
        off_v = lax.fori_loop(0, NA_PAD // (2 * L), comp_body, zf)

        k_v = jnp.minimum(off_v.astype(jnp.int32), K_MAX)
        ngU_v = (k_v + (U * L - 1)) >> 6
        ngU = ngU_v[0]

        def gather_body(g, _):
            idxv = can[pl.ds(g * L, L)]
            cx1[pl.ds(g * L, L)] = plsc.load_gather(x1_v, [idxv])
            cy1[pl.ds(g * L, L)] = plsc.load_gather(y1_v, [idxv])
            cx2[pl.ds(g * L, L)] = plsc.load_gather(x2_v, [idxv])
            cy2[pl.ds(g * L, L)] = plsc.load_gather(y2_v, [idxv])
            car[pl.ds(g * L, L)] = plsc.load_gather(ar_v, [idxv])
            return 0
        lax.fori_loop(0, U * ngU, gather_body, 0)

        def init_body(g, carry):
            lms, lgs = carry
            lms = list(lms)
            lgs = list(lgs)
            vs = [csc[pl.ds((U * g + u) * L, L)] for u in range(U)]
            for u in range(U):
                gf = jnp.broadcast_to(U * g + u, (L,)).astype(jnp.float32)
                upd = vs[u] > lms[u]
                lms[u] = jnp.where(upd, vs[u], lms[u])
                lgs[u] = jnp.where(upd, gf, lgs[u])
            return tuple(lms), tuple(lgs)
        carry0 = lax.fori_loop(0, ngU, init_body,
                               ((zf,) * U, (zf,) * U))

        def pick_body(i, carry):
            lms, lgs = carry
            m01 = jnp.maximum(lms[0], lms[1])
            m23 = jnp.maximum(lms[2], lms[3])
            gmax = _splat_max(jnp.maximum(m01, m23), ji)
            alive = gmax > 0.0
            bigv = jnp.broadcast_to(jnp.float32(BIGF), (L,))
            cm = bigv
            for u in range(U):
                cm = jnp.minimum(
                    cm, jnp.where(lms[u] == gmax, lgs[u] * L + jf, bigv))
            posf = _splat_min(cm, ji)
            posi = jnp.where(alive, posf, 0.0).astype(jnp.int32)

            anchor_v = plsc.load_gather(can, [posi])
            px1 = plsc.load_gather(cx1, [posi])
            py1 = plsc.load_gather(cy1, [posi])
            px2 = plsc.load_gather(cx2, [posi])
            py2 = plsc.load_gather(cy2, [posi])
            par = plsc.load_gather(car, [posi])

            ivec = jnp.broadcast_to(i, (L,))
            plsc.store_scatter(osc, [ivec],
                               jnp.where(alive, gmax, 0.0), mask=lane0)
            plsc.store_scatter(oan, [ivec],
                               jnp.where(alive, anchor_v, 0), mask=lane0)

            def supp_body(g, carry2):
                lms, lgs = carry2
                lms = list(lms)
                lgs = list(lgs)
                sls = [pl.ds((U * g + u) * L, L) for u in range(U)]
                vs = [csc[s] for s in sls]
                xx1 = [cx1[s] for s in sls]
                yy1 = [cy1[s] for s in sls]
                xx2 = [cx2[s] for s in sls]
                yy2 = [cy2[s] for s in sls]
                ars = [car[s] for s in sls]
                ws = [jnp.maximum(jnp.minimum(px2, xx2[u])
                                  - jnp.maximum(px1, xx1[u]), 0.0)
                      for u in range(U)]
                hs = [jnp.maximum(jnp.minimum(py2, yy2[u])
                                  - jnp.maximum(py1, yy1[u]), 0.0)
                      for u in range(U)]
                ins = [ws[u] * hs[u] for u in range(U)]
                ious = [ins[u] / (((par + ars[u]) - ins[u]) + 1e-8)
                        for u in range(U)]
                v2s = []
                for u in range(U):
                    kill = (((ious[u] >= CRITERIA) & alive)
                            | ((U * g + u) * L + ji == posi))
                    v2s.append(jnp.where(kill, 0.0, vs[u]))
                for u in range(U):
                    csc[sls[u]] = v2s[u]
                for u in range(U):
                    gf = jnp.broadcast_to(U * g + u, (L,)).astype(jnp.float32)
                    upd = v2s[u] > lms[u]
                    lms[u] = jnp.where(upd, v2s[u], lms[u])
                    lgs[u] = jnp.where(upd, gf, lgs[u])
                return tuple(lms), tuple(lgs)
            return lax.fori_loop(0, ngU, supp_body,
                                 ((zf,) * U, (zf,) * U))
        lax.fori_loop(0, MAX_OUT, pick_body, carry0)

        pltpu.sync_copy(osc, psc_hbm.at[c])
        pltpu.sync_copy(oan, pan_hbm.at[c])

    for k in range(3):
        c = wid + k * NW

        @pl.when(c < CM)
        def _():
            process_class(c)


def _sc_merge(psc_hbm, pan_hbm, geom_hbm, obox_hbm, olab_hbm, osc_hbm,
              psc_v, pan_v, x1_v, y1_v, x2_v, y2_v,
              heads_v, ptr_v, obox_v, olab_v, osc_v):
    wid = lax.axis_index("s") * 2 + lax.axis_index("c")
    ji = lax.broadcasted_iota(jnp.int32, (L,), 0)
    jf = ji.astype(jnp.float32)
    lane0 = ji == 0
    NG = 6

    @pl.when(wid == 0)
    def _():
        pltpu.sync_copy(psc_hbm, psc_v)
        pltpu.sync_copy(pan_hbm, pan_v)
        pltpu.sync_copy(geom_hbm.at[0], x1_v)
        pltpu.sync_copy(geom_hbm.at[1], y1_v)
        pltpu.sync_copy(geom_hbm.at[2], x2_v)
        pltpu.sync_copy(geom_hbm.at[3], y2_v)

        zi = jnp.zeros((L,), jnp.int32)
        for g in range(NG):
            cls = g * L + ji
            valid = cls < CM
            clsc = jnp.minimum(cls, CM - 1)
            h = plsc.load_gather(psc_v, [clsc, zi])
            heads_v[pl.ds(g * L, L)] = jnp.where(
                valid, h, jnp.broadcast_to(jnp.float32(NEG), (L,)))
            ptr_v[pl.ds(g * L, L)] = zi

        def merge_body(i, _):
            lanemax = jnp.broadcast_to(jnp.float32(NEG), (L,))
            lanecls = jnp.zeros((L,), jnp.float32)
            for g in range(NG):
                v = heads_v[pl.ds(g * L, L)]
                upd = v > lanemax
                lanemax = jnp.where(upd, v, lanemax)
                lanecls = jnp.where(upd, g * L + jf, lanecls)
            gmax = _splat_max(lanemax, ji)
            candcls = jnp.where(lanemax == gmax, lanecls,
                                jnp.broadcast_to(jnp.float32(BIGF), (L,)))
            clsi = _splat_min(candcls, ji).astype(jnp.int32)

            p_v = plsc.load_gather(ptr_v, [clsi])
            anchor_v = plsc.load_gather(pan_v, [clsi, p_v])
            bx1 = plsc.load_gather(x1_v, [anchor_v])
            by1 = plsc.load_gather(y1_v, [anchor_v])
            bx2 = plsc.load_gather(x2_v, [anchor_v])
            by2 = plsc.load_gather(y2_v, [anchor_v])

            bval = jnp.where(ji == 0, bx1,
                             jnp.where(ji == 1, by1,
                                       jnp.where(ji == 2, bx2, by2)))
            plsc.store_scatter(obox_v, [4 * i + ji], bval, mask=ji < 4)
            ivec = jnp.broadcast_to(i, (L,))
            plsc.store_scatter(olab_v, [ivec], clsi + 1, mask=lane0)
            plsc.store_scatter(osc_v, [ivec], gmax, mask=lane0)

            p1 = p_v + 1
            plsc.store_scatter(ptr_v, [clsi], p1, mask=lane0)
            nh = plsc.load_gather(
                psc_v, [clsi, jnp.minimum(p1, MAX_OUT - 1)])
            nh = jnp.where(p1 < MAX_OUT, nh,
                           jnp.broadcast_to(jnp.float32(NEG), (L,)))
            plsc.store_scatter(heads_v, [clsi], nh, mask=lane0)
            return 0
        lax.fori_loop(0, MAX_OUT, merge_body, 0)

        pltpu.sync_copy(obox_v, obox_hbm)
        pltpu.sync_copy(olab_v, olab_hbm)
        pltpu.sync_copy(osc_v, osc_hbm)


def kernel(ploc, plabel, dboxes):
    assert ploc.shape == (1, 4, NA) and plabel.shape == (1, NCLS, NA)
    pad = NA_PAD - NA
    plabel_p = jnp.pad(plabel[0], ((0, 0), (0, pad)))
    ploc_p = jnp.pad(ploc[0], ((0, 0), (0, pad)))
    dbox_p = jnp.pad(dboxes.T, ((0, 0), (0, pad)))

    probs, geom = pl.pallas_call(
        _tc_dense,
        out_shape=(
            jax.ShapeDtypeStruct((CM, NA_PAD), jnp.float32),
            jax.ShapeDtypeStruct((8, NA_PAD), jnp.float32),
        ),
    )(plabel_p, ploc_p, dbox_p)

    mesh = plsc.VectorSubcoreMesh(core_axis_name="c", subcore_axis_name="s",
                                  num_cores=2, num_subcores=16)

    nms = pl.kernel(
        _sc_nms, mesh=mesh,
        compiler_params=pltpu.CompilerParams(needs_layout_passes=False),
        out_type=(
            jax.ShapeDtypeStruct((CM, PICK_PAD), jnp.float32),
            jax.ShapeDtypeStruct((CM, PICK_PAD), jnp.int32),
        ),
        scratch_types=[
            pltpu.VMEM((NA_PAD,), jnp.float32),
            pltpu.VMEM((NA_PAD,), jnp.float32),
            pltpu.VMEM((NA_PAD,), jnp.float32),
            pltpu.VMEM((NA_PAD,), jnp.float32),
            pltpu.VMEM((NA_PAD,), jnp.float32),
            pltpu.VMEM((NA_PAD,), jnp.float32),
            pltpu.VMEM((K_MAX + 4 * L,), jnp.float32),
            pltpu.VMEM((K_MAX + 4 * L,), jnp.int32),
            pltpu.VMEM((K_MAX + 4 * L,), jnp.float32),
            pltpu.VMEM((K_MAX + 4 * L,), jnp.float32),
            pltpu.VMEM((K_MAX + 4 * L,), jnp.float32),
            pltpu.VMEM((K_MAX + 4 * L,), jnp.float32),
            pltpu.VMEM((K_MAX + 4 * L,), jnp.float32),
            pltpu.VMEM((PICK_PAD,), jnp.float32),
            pltpu.VMEM((PICK_PAD,), jnp.int32),
        ],
    )
    pick_sc, pick_an = nms(probs, geom)

    merge = pl.kernel(
        _sc_merge, mesh=mesh,
        compiler_params=pltpu.CompilerParams(needs_layout_passes=False),
        out_type=(
            jax.ShapeDtypeStruct((4 * MAX_OUT,), jnp.float32),
            jax.ShapeDtypeStruct((MAX_OUT,), jnp.int32),
            jax.ShapeDtypeStruct((MAX_OUT,), jnp.float32),
        ),
        scratch_types=[
            pltpu.VMEM((CM, PICK_PAD), jnp.float32),
            pltpu.VMEM((CM, PICK_PAD), jnp.int32),
            pltpu.VMEM((NA_PAD,), jnp.float32),
            pltpu.VMEM((NA_PAD,), jnp.float32),
            pltpu.VMEM((NA_PAD,), jnp.float32),
            pltpu.VMEM((NA_PAD,), jnp.float32),
            pltpu.VMEM((96,), jnp.float32),
            pltpu.VMEM((96,), jnp.int32),
            pltpu.VMEM((4 * MAX_OUT,), jnp.float32),
            pltpu.VMEM((MAX_OUT,), jnp.int32),
            pltpu.VMEM((MAX_OUT,), jnp.float32),
        ],
    )
    obox, olab, osc = merge(pick_sc, pick_an, geom)

    return (obox.reshape(1, MAX_OUT, 4), olab.reshape(1, MAX_OUT),
            osc.reshape(1, MAX_OUT))
